# baseline (speedup 1.0000x reference)
import functools

import jax
import jax.numpy as jnp
from jax import lax
from jax.experimental import pallas as pl
from jax.experimental.pallas import tpu as pltpu
from jax.experimental.pallas import tpu_sc as plsc

_N = 2048
_D = 64
_KP1 = 17
_BLK = 256
_KEY_MASK = ~2047
_BIG = 0x7FFFFFFF

_NC = 2
_NS = 16
_NW = _NC * _NS
_RPW = _N // _NW


def _tc_pack_body(pos_ref, out_ref):
    i = pl.program_id(0)
    p = pos_ref[...]
    a = pos_ref[pl.ds(i * _BLK, _BLK), :]
    g = lax.dot_general(a, p, (((1,), (1,)), ((), ())),
                        preferred_element_type=jnp.float32)
    na = jnp.sum(a * a, axis=1, keepdims=True)
    nb = jnp.sum(p * p, axis=1)[None, :]
    s = jnp.maximum(na + nb - 2.0 * g, 0.0)
    col = lax.broadcasted_iota(jnp.int32, s.shape, 1)
    row = lax.broadcasted_iota(jnp.int32, s.shape, 0) + i * _BLK
    s = jnp.where(col == row, 0.0, s)
    nrm = jnp.sqrt(s)
    bits = lax.bitcast_convert_type(nrm, jnp.int32)
    out_ref[...] = (bits & jnp.int32(_KEY_MASK)) | col


def _tc_pack(positions):
    return pl.pallas_call(
        _tc_pack_body,
        grid=(_N // _BLK,),
        in_specs=[
            pl.BlockSpec((_N, _D), lambda i: (0, 0)),
        ],
        out_specs=pl.BlockSpec((_BLK, _N), lambda i: (i, 0)),
        out_shape=jax.ShapeDtypeStruct((_N, _N), jnp.int32),
    )(positions)


def _merge_step(r, x, chunk):
    cd = plsc.sort_key_val(chunk, chunk, descending=True)[0]
    m = jnp.minimum(r, cd)
    xc = jnp.maximum(r, cd)
    return jnp.sort(m), jnp.minimum(x, xc)


def _sc_topk_body(keys_hbm, out_hbm, buf0, buf1, ob, sem0, sem1):
    wid = lax.axis_index("s") * _NC + lax.axis_index("c")
    base_row = wid * _RPW
    gather_off = lax.iota(jnp.int32, 16) * 16
    big16 = jnp.full((16,), _BIG, jnp.int32)

    def row_topk(buf, row, slot):
        groups = []
        for g in range(8):
            acc = buf[row, pl.ds(g * 256, 16)]
            for i in range(1, 16):
                acc = jnp.minimum(acc, buf[row, pl.ds(g * 256 + i * 16, 16)])
            groups.append(acc)
        r = jnp.sort(groups[0])
        x = big16
        for g in range(1, 8):
            r, x = _merge_step(r, x, groups[g])
        d = jnp.min(x)
        row16 = jnp.broadcast_to(row, (16,))
        def block_chunk(key_scalar):
            c = key_scalar & 2047
            base = lax.shift_right_logical(c, 8) * 256 + (c & 15)
            return plsc.load_gather(buf, [row16, gather_off + base])

        r2 = jnp.sort(block_chunk(r[0]))
        x2 = big16
        for j in range(1, 16):
            r2, x2 = _merge_step(r2, x2, block_chunk(r[j]))
        r2, x2 = _merge_step(r2, x2, block_chunk(d))
        d2 = jnp.min(x2)
        km = jnp.int32(_KEY_MASK)
        ob[pl.ds(slot * _KP1 + 16, 16)] = plsc.bitcast(
            jnp.broadcast_to(d2 & km, (16,)), jnp.float32)
        ob[pl.ds(slot * _KP1, 16)] = plsc.bitcast(r2 & km, jnp.float32)

    rb = 8

    def wait_batch(sem):
        pltpu.make_async_copy(keys_hbm.at[pl.ds(0, rb)], buf0, sem).wait()

    pltpu.async_copy(keys_hbm.at[pl.ds(base_row, rb)], buf0, sem0)

    def outer(i, carry):
        r0 = base_row + i * (2 * rb)

        def inner0(j, c):
            row_topk(buf0, j, i * (2 * rb) + j)
            return c

        def inner1(j, c):
            row_topk(buf1, j, i * (2 * rb) + rb + j)
            return c

        wait_batch(sem0)
        pltpu.async_copy(keys_hbm.at[pl.ds(r0 + rb, rb)], buf1, sem1)
        lax.fori_loop(0, rb, inner0, carry)
        wait_batch(sem1)
        nxt = jnp.minimum(r0 + 2 * rb, _N - rb)
        pltpu.async_copy(keys_hbm.at[pl.ds(nxt, rb)], buf0, sem0)
        lax.fori_loop(0, rb, inner1, carry)
        return carry

    lax.fori_loop(0, _RPW // (2 * rb), outer, jnp.int32(0))
    wait_batch(sem0)
    ow = _RPW * _KP1
    pltpu.sync_copy(ob.at[pl.ds(0, ow)], out_hbm.at[pl.ds(wid * ow, ow)])


def _sc_topk(keys):
    mesh = plsc.VectorSubcoreMesh(core_axis_name="c", subcore_axis_name="s",
                                  num_cores=_NC, num_subcores=_NS)
    f = pl.kernel(
        _sc_topk_body,
        out_type=jax.ShapeDtypeStruct((_N * _KP1,), jnp.float32),
        mesh=mesh,
        compiler_params=pltpu.CompilerParams(needs_layout_passes=False),
        scratch_types=[
            pltpu.VMEM((8, _N), jnp.int32),
            pltpu.VMEM((8, _N), jnp.int32),
            pltpu.VMEM((_RPW * _KP1 + 16,), jnp.float32),
            pltpu.SemaphoreType.DMA,
            pltpu.SemaphoreType.DMA,
        ],
    )
    return f(keys)


def kernel(positions, k):
    del k
    keys = _tc_pack(positions)
    out = _sc_topk(keys)
    return out.reshape(_N, _KP1)


# speedup vs baseline: 1.0817x; 1.0817x over previous
import functools

import jax
import jax.numpy as jnp
from jax import lax
from jax.experimental import pallas as pl
from jax.experimental.pallas import tpu as pltpu
from jax.experimental.pallas import tpu_sc as plsc

_N = 2048
_D = 64
_KP1 = 17
_BLK = 256
_KEY_MASK = ~2047
_BIG = 0x7FFFFFFF

_NC = 2
_NS = 16
_NW = _NC * _NS
_RPW = _N // _NW


def _tc_pack_body(pos_ref, out_ref):
    i = pl.program_id(0)
    p = pos_ref[...]
    a = pos_ref[pl.ds(i * _BLK, _BLK), :]
    g = lax.dot_general(a, p, (((1,), (1,)), ((), ())),
                        preferred_element_type=jnp.float32)
    na = jnp.sum(a * a, axis=1, keepdims=True)
    nb = jnp.sum(p * p, axis=1)[None, :]
    s = jnp.maximum(na + nb - 2.0 * g, 0.0)
    col = lax.broadcasted_iota(jnp.int32, s.shape, 1)
    row = lax.broadcasted_iota(jnp.int32, s.shape, 0) + i * _BLK
    s = jnp.where(col == row, 0.0, s)
    nrm = jnp.sqrt(s)
    bits = lax.bitcast_convert_type(nrm, jnp.int32)
    out_ref[...] = (bits & jnp.int32(_KEY_MASK)) | col


def _tc_pack(positions):
    return pl.pallas_call(
        _tc_pack_body,
        grid=(_N // _BLK,),
        in_specs=[
            pl.BlockSpec((_N, _D), lambda i: (0, 0)),
        ],
        out_specs=pl.BlockSpec((_BLK, _N), lambda i: (i, 0)),
        out_shape=jax.ShapeDtypeStruct((_N, _N), jnp.int32),
    )(positions)


def _tree_min(vs):
    while len(vs) > 1:
        nxt = [jnp.minimum(vs[i], vs[i + 1]) for i in range(0, len(vs) - 1, 2)]
        if len(vs) % 2:
            nxt.append(vs[-1])
        vs = nxt
    return vs[0]


def _merge2(ra, da, rb, db):
    cd = lax.rev(rb, (0,))
    m = jnp.minimum(ra, cd)
    x = jnp.maximum(ra, cd)
    return jnp.sort(m), jnp.minimum(jnp.minimum(da, db), jnp.min(x))


def _merge_tree(sorted_chunks):
    big = jnp.int32(_BIG)
    lvl = [(c, big) for c in sorted_chunks]
    while len(lvl) > 1:
        nxt = [_merge2(*lvl[i], *lvl[i + 1]) for i in range(0, len(lvl) - 1, 2)]
        if len(lvl) % 2:
            nxt.append(lvl[-1])
        lvl = nxt
    return lvl[0]


def _sc_topk_body(keys_hbm, out_hbm, buf0, buf1, ob, sem0, sem1):
    wid = lax.axis_index("s") * _NC + lax.axis_index("c")
    base_row = wid * _RPW
    gather_off = lax.iota(jnp.int32, 16) * 16
    big16 = jnp.full((16,), _BIG, jnp.int32)

    def row_topk(buf, row, slot):
        groups = []
        for g in range(8):
            chunks = [buf[row, pl.ds(g * 256 + i * 16, 16)] for i in range(16)]
            groups.append(_tree_min(chunks))
        r, d = _merge_tree([jnp.sort(g) for g in groups])
        row16 = jnp.broadcast_to(row, (16,))
        def block_chunk(key_scalar):
            c = key_scalar & 2047
            base = lax.shift_right_logical(c, 8) * 256 + (c & 15)
            return plsc.load_gather(buf, [row16, gather_off + base])

        cands = [jnp.sort(block_chunk(r[j])) for j in range(16)]
        cands.append(jnp.sort(block_chunk(d)))
        r2, d2 = _merge_tree(cands)
        km = jnp.int32(_KEY_MASK)
        ob[pl.ds(slot * _KP1 + 16, 16)] = plsc.bitcast(
            jnp.broadcast_to(d2 & km, (16,)), jnp.float32)
        ob[pl.ds(slot * _KP1, 16)] = plsc.bitcast(r2 & km, jnp.float32)

    rb = 8

    def wait_batch(sem):
        pltpu.make_async_copy(keys_hbm.at[pl.ds(0, rb)], buf0, sem).wait()

    pltpu.async_copy(keys_hbm.at[pl.ds(base_row, rb)], buf0, sem0)

    def outer(i, carry):
        r0 = base_row + i * (2 * rb)

        def inner0(j, c):
            row_topk(buf0, 2 * j, i * (2 * rb) + 2 * j)
            row_topk(buf0, 2 * j + 1, i * (2 * rb) + 2 * j + 1)
            return c

        def inner1(j, c):
            row_topk(buf1, 2 * j, i * (2 * rb) + rb + 2 * j)
            row_topk(buf1, 2 * j + 1, i * (2 * rb) + rb + 2 * j + 1)
            return c

        wait_batch(sem0)
        pltpu.async_copy(keys_hbm.at[pl.ds(r0 + rb, rb)], buf1, sem1)
        lax.fori_loop(0, rb // 2, inner0, carry)
        wait_batch(sem1)
        nxt = jnp.minimum(r0 + 2 * rb, _N - rb)
        pltpu.async_copy(keys_hbm.at[pl.ds(nxt, rb)], buf0, sem0)
        lax.fori_loop(0, rb // 2, inner1, carry)
        return carry

    lax.fori_loop(0, _RPW // (2 * rb), outer, jnp.int32(0))
    wait_batch(sem0)
    ow = _RPW * _KP1
    pltpu.sync_copy(ob.at[pl.ds(0, ow)], out_hbm.at[pl.ds(wid * ow, ow)])


def _sc_topk(keys):
    mesh = plsc.VectorSubcoreMesh(core_axis_name="c", subcore_axis_name="s",
                                  num_cores=_NC, num_subcores=_NS)
    f = pl.kernel(
        _sc_topk_body,
        out_type=jax.ShapeDtypeStruct((_N * _KP1,), jnp.float32),
        mesh=mesh,
        compiler_params=pltpu.CompilerParams(needs_layout_passes=False),
        scratch_types=[
            pltpu.VMEM((8, _N), jnp.int32),
            pltpu.VMEM((8, _N), jnp.int32),
            pltpu.VMEM((_RPW * _KP1 + 16,), jnp.float32),
            pltpu.SemaphoreType.DMA,
            pltpu.SemaphoreType.DMA,
        ],
    )
    return f(keys)


def kernel(positions, k):
    del k
    keys = _tc_pack(positions)
    out = _sc_topk(keys)
    return out.reshape(_N, _KP1)
